# baseline (speedup 1.0000x reference)
import functools

import jax
import jax.numpy as jnp
from jax import lax
from jax.experimental import pallas as pl
from jax.experimental.pallas import tpu as pltpu
from jax.experimental.pallas import tpu_sc as plsc

N_NODES = 10000
N_EDGES = 320000
D = 128

NC = 2
NS = 16
CHUNK = 104
EDGES_PER_SC = N_EDGES // NC
EDGES_PER_TILE = EDGES_PER_SC // NS
N_CHUNKS = EDGES_PER_TILE // CHUNK
TAIL = EDGES_PER_TILE - N_CHUNKS * CHUNK
N_PAD = 10240
ROWS_PER_TILE = N_PAD // NS


NBUF = 3
N_GROUPS = N_CHUNKS // NBUF


@functools.partial(
    pl.kernel,
    out_type=jax.ShapeDtypeStruct((NC, N_PAD, D), jnp.float32),
    mesh=plsc.VectorSubcoreMesh(core_axis_name="c", subcore_axis_name="s"),
    scratch_types=[
        pltpu.VMEM((NBUF, 2, CHUNK), jnp.int32),
        pltpu.VMEM((2, TAIL), jnp.int32),
        pltpu.VMEM((NBUF, CHUNK, D), jnp.float32),
        pltpu.VMEM_SHARED((N_PAD, D), jnp.float32),
        pltpu.SemaphoreType.DMA((NBUF,)),
        pltpu.SemaphoreType.DMA((NBUF,)),
        pltpu.SemaphoreType.DMA((NBUF,)),
    ],
)
def _spmm(h_hbm, idx4_hbm, idxt_hbm, out_hbm,
          idx_v, tidx_v, rows_v, acc_sh, isem, gsem, ssem):
    c = lax.axis_index("c")
    s = lax.axis_index("s")
    w = c * NS + s
    r0 = s * ROWS_PER_TILE

    def idx_start(j, b):
        pltpu.async_copy(idx4_hbm.at[w, j], idx_v.at[b], isem.at[b])

    def idx_wait(j, b):
        pltpu.make_async_copy(idx4_hbm.at[w, j], idx_v.at[b],
                              isem.at[b]).wait()

    def gather_start(b):
        pltpu.async_copy(h_hbm.at[idx_v.at[b, 0]], rows_v.at[b], gsem.at[b])

    def gather_wait(b):
        pltpu.make_async_copy(h_hbm.at[idx_v.at[b, 0]], rows_v.at[b],
                              gsem.at[b]).wait()

    def scatter_start(b):
        pltpu.async_copy(rows_v.at[b], acc_sh.at[idx_v.at[b, 1]], ssem.at[b],
                         add=True)

    def scatter_wait(b):
        pltpu.make_async_copy(rows_v.at[b], acc_sh.at[idx_v.at[b, 1]],
                              ssem.at[b]).wait()

    for b in range(NBUF):
        idx_start(b, b)
    for i in range(CHUNK):
        for k in range(D // 16):
            rows_v[0, i, pl.ds(k * 16, 16)] = jnp.zeros((16,), jnp.float32)
    zcopies = [
        pltpu.make_async_copy(
            rows_v.at[0], acc_sh.at[pl.ds(r0 + t * CHUNK, CHUNK)], ssem.at[0])
        for t in range(ROWS_PER_TILE // CHUNK)
    ] + [
        pltpu.make_async_copy(
            rows_v.at[0, pl.ds(0, ROWS_PER_TILE % CHUNK)],
            acc_sh.at[pl.ds(r0 + ROWS_PER_TILE // CHUNK * CHUNK,
                            ROWS_PER_TILE % CHUNK)],
            ssem.at[0])
    ]
    for zc in zcopies:
        zc.start()
    for zc in zcopies:
        zc.wait()
    plsc.subcore_barrier()

    def group(g, carry):
        for b in range(NBUF):
            idx_wait(g * NBUF + b, b)
            gather_start(b)
        for b in range(NBUF):
            gather_wait(b)
            scatter_start(b)
        for b in range(NBUF):
            scatter_wait(b)
            idx_start(g * NBUF + b + NBUF, b)
        return carry

    lax.fori_loop(0, N_GROUPS - 1, group, 0)

    gl = N_GROUPS - 1
    for b in range(NBUF):
        idx_wait(gl * NBUF + b, b)
        gather_start(b)
    for b in range(NBUF):
        gather_wait(b)
        scatter_start(b)
    for b in range(NBUF):
        scatter_wait(b)

    pltpu.async_copy(idxt_hbm.at[w], tidx_v, isem.at[0]).wait()
    tgather = pltpu.make_async_copy(
        h_hbm.at[tidx_v.at[0]], rows_v.at[0, pl.ds(0, TAIL)], gsem.at[0])
    tgather.start()
    tgather.wait()
    tscatter = pltpu.make_async_copy(
        rows_v.at[0, pl.ds(0, TAIL)], acc_sh.at[tidx_v.at[1]], ssem.at[0])
    tscatter.start(add=True)
    tscatter.wait()

    plsc.subcore_barrier()
    pltpu.sync_copy(acc_sh.at[pl.ds(r0, ROWS_PER_TILE)],
                    out_hbm.at[c, pl.ds(r0, ROWS_PER_TILE)])


def _mm_body(x_ref, w_ref, o_ref):
    o_ref[...] = jnp.dot(x_ref[...], w_ref[...],
                         preferred_element_type=jnp.float32)


_mm = pl.pallas_call(
    _mm_body,
    out_shape=jax.ShapeDtypeStruct((N_NODES, D), jnp.float32),
)


def _addrelu_body(p_ref, o_ref):
    o_ref[...] = jnp.maximum(p_ref[0] + p_ref[1], 0.0)


_addrelu = pl.pallas_call(
    _addrelu_body,
    out_shape=jax.ShapeDtypeStruct((N_PAD, D), jnp.float32),
)


def _final_body(p_ref, w_ref, o_ref):
    p = p_ref[0, pl.ds(0, N_NODES)] + p_ref[1, pl.ds(0, N_NODES)]
    g = jnp.dot(p, w_ref[...], preferred_element_type=jnp.float32)
    m = jnp.max(g, axis=1, keepdims=True)
    o_ref[...] = (g - m) - jnp.log(jnp.sum(jnp.exp(g - m), axis=1,
                                           keepdims=True))


_final = pl.pallas_call(
    _final_body,
    out_shape=jax.ShapeDtypeStruct((N_NODES, D), jnp.float32),
)


def kernel(x, edge_index, W1, W2):
    src2 = edge_index[1].astype(jnp.int32).reshape(NC * NS, EDGES_PER_TILE)
    dst2 = edge_index[0].astype(jnp.int32).reshape(NC * NS, EDGES_PER_TILE)
    nm = N_CHUNKS * CHUNK
    idx4 = jnp.stack([src2[:, :nm].reshape(NC * NS, N_CHUNKS, CHUNK),
                      dst2[:, :nm].reshape(NC * NS, N_CHUNKS, CHUNK)],
                     axis=2)
    idxt = jnp.stack([src2[:, nm:], dst2[:, nm:]], axis=1)
    h1 = _mm(x, W1)
    p1 = _spmm(h1, idx4, idxt)
    r = _addrelu(p1)
    p2 = _spmm(r, idx4, idxt)
    return _final(p2, W2)


# speedup vs baseline: 1.0156x; 1.0156x over previous
import functools

import jax
import jax.numpy as jnp
from jax import lax
from jax.experimental import pallas as pl
from jax.experimental.pallas import tpu as pltpu
from jax.experimental.pallas import tpu_sc as plsc

N_NODES = 10000
N_EDGES = 320000
D = 128

NC = 2
NS = 16
CHUNK = 64
EDGES_PER_SC = N_EDGES // NC
EDGES_PER_TILE = EDGES_PER_SC // NS
N_CHUNKS = EDGES_PER_TILE // CHUNK
TAIL = EDGES_PER_TILE - N_CHUNKS * CHUNK
N_PAD = 10240
ROWS_PER_TILE = N_PAD // NS


NBUF = 5
N_GROUPS = N_CHUNKS // NBUF


@functools.partial(
    pl.kernel,
    out_type=jax.ShapeDtypeStruct((NC, N_PAD, D), jnp.float32),
    mesh=plsc.VectorSubcoreMesh(core_axis_name="c", subcore_axis_name="s"),
    scratch_types=[
        pltpu.VMEM((NBUF, 2, CHUNK), jnp.int32),
        pltpu.VMEM((2, TAIL), jnp.int32),
        pltpu.VMEM((NBUF, CHUNK, D), jnp.float32),
        pltpu.VMEM_SHARED((N_PAD, D), jnp.float32),
        pltpu.SemaphoreType.DMA((NBUF,)),
        pltpu.SemaphoreType.DMA((NBUF,)),
        pltpu.SemaphoreType.DMA((NBUF,)),
    ],
)
def _spmm(h_hbm, idx4_hbm, idxt_hbm, out_hbm,
          idx_v, tidx_v, rows_v, acc_sh, isem, gsem, ssem):
    c = lax.axis_index("c")
    s = lax.axis_index("s")
    w = c * NS + s
    r0 = s * ROWS_PER_TILE

    def idx_start(j, b):
        pltpu.async_copy(idx4_hbm.at[w, j], idx_v.at[b], isem.at[b])

    def idx_wait(j, b):
        pltpu.make_async_copy(idx4_hbm.at[w, j], idx_v.at[b],
                              isem.at[b]).wait()

    def gather_start(b):
        pltpu.async_copy(h_hbm.at[idx_v.at[b, 0]], rows_v.at[b], gsem.at[b])

    def gather_wait(b):
        pltpu.make_async_copy(h_hbm.at[idx_v.at[b, 0]], rows_v.at[b],
                              gsem.at[b]).wait()

    def scatter_start(b):
        pltpu.async_copy(rows_v.at[b], acc_sh.at[idx_v.at[b, 1]], ssem.at[b],
                         add=True)

    def scatter_wait(b):
        pltpu.make_async_copy(rows_v.at[b], acc_sh.at[idx_v.at[b, 1]],
                              ssem.at[b]).wait()

    for b in range(NBUF):
        idx_start(b, b)
    for i in range(CHUNK):
        for k in range(D // 16):
            rows_v[0, i, pl.ds(k * 16, 16)] = jnp.zeros((16,), jnp.float32)
    zcopies = [
        pltpu.make_async_copy(
            rows_v.at[0], acc_sh.at[pl.ds(r0 + t * CHUNK, CHUNK)], ssem.at[0])
        for t in range(ROWS_PER_TILE // CHUNK)
    ] + [
        pltpu.make_async_copy(
            rows_v.at[0, pl.ds(0, ROWS_PER_TILE % CHUNK)],
            acc_sh.at[pl.ds(r0 + ROWS_PER_TILE // CHUNK * CHUNK,
                            ROWS_PER_TILE % CHUNK)],
            ssem.at[0])
    ]
    for zc in zcopies:
        zc.start()
    for zc in zcopies:
        zc.wait()
    plsc.subcore_barrier()

    def group(g, carry):
        for b in range(NBUF):
            idx_wait(g * NBUF + b, b)
            gather_start(b)
        for b in range(NBUF):
            gather_wait(b)
            scatter_start(b)
        for b in range(NBUF):
            scatter_wait(b)
            idx_start(g * NBUF + b + NBUF, b)
        return carry

    lax.fori_loop(0, N_GROUPS - 1, group, 0)

    gl = N_GROUPS - 1
    for b in range(NBUF):
        idx_wait(gl * NBUF + b, b)
        gather_start(b)
    for b in range(NBUF):
        gather_wait(b)
        scatter_start(b)
    for b in range(NBUF):
        scatter_wait(b)

    for j in range(N_GROUPS * NBUF, N_CHUNKS):
        idx_start(j, 0)
        idx_wait(j, 0)
        gather_start(0)
        gather_wait(0)
        scatter_start(0)
        scatter_wait(0)

    pltpu.async_copy(idxt_hbm.at[w], tidx_v, isem.at[0]).wait()
    tgather = pltpu.make_async_copy(
        h_hbm.at[tidx_v.at[0]], rows_v.at[0, pl.ds(0, TAIL)], gsem.at[0])
    tgather.start()
    tgather.wait()
    tscatter = pltpu.make_async_copy(
        rows_v.at[0, pl.ds(0, TAIL)], acc_sh.at[tidx_v.at[1]], ssem.at[0])
    tscatter.start(add=True)
    tscatter.wait()

    plsc.subcore_barrier()
    pltpu.sync_copy(acc_sh.at[pl.ds(r0, ROWS_PER_TILE)],
                    out_hbm.at[c, pl.ds(r0, ROWS_PER_TILE)])


def _mm_body(x_ref, w_ref, o_ref):
    o_ref[...] = jnp.dot(x_ref[...], w_ref[...],
                         preferred_element_type=jnp.float32)


_mm = pl.pallas_call(
    _mm_body,
    out_shape=jax.ShapeDtypeStruct((N_NODES, D), jnp.float32),
)


def _addrelu_body(p_ref, o_ref):
    o_ref[...] = jnp.maximum(p_ref[0] + p_ref[1], 0.0)


_addrelu = pl.pallas_call(
    _addrelu_body,
    out_shape=jax.ShapeDtypeStruct((N_PAD, D), jnp.float32),
)


def _final_body(p_ref, w_ref, o_ref):
    p = p_ref[0, pl.ds(0, N_NODES)] + p_ref[1, pl.ds(0, N_NODES)]
    g = jnp.dot(p, w_ref[...], preferred_element_type=jnp.float32)
    m = jnp.max(g, axis=1, keepdims=True)
    o_ref[...] = (g - m) - jnp.log(jnp.sum(jnp.exp(g - m), axis=1,
                                           keepdims=True))


_final = pl.pallas_call(
    _final_body,
    out_shape=jax.ShapeDtypeStruct((N_NODES, D), jnp.float32),
)


def kernel(x, edge_index, W1, W2):
    src2 = edge_index[1].astype(jnp.int32).reshape(NC * NS, EDGES_PER_TILE)
    dst2 = edge_index[0].astype(jnp.int32).reshape(NC * NS, EDGES_PER_TILE)
    nm = N_CHUNKS * CHUNK
    idx4 = jnp.stack([src2[:, :nm].reshape(NC * NS, N_CHUNKS, CHUNK),
                      dst2[:, :nm].reshape(NC * NS, N_CHUNKS, CHUNK)],
                     axis=2)
    idxt = jnp.stack([src2[:, nm:], dst2[:, nm:]], axis=1)
    h1 = _mm(x, W1)
    p1 = _spmm(h1, idx4, idxt)
    r = _addrelu(p1)
    p2 = _spmm(r, idx4, idxt)
    return _final(p2, W2)


# speedup vs baseline: 1.1026x; 1.0857x over previous
import functools

import jax
import jax.numpy as jnp
from jax import lax
from jax.experimental import pallas as pl
from jax.experimental.pallas import tpu as pltpu
from jax.experimental.pallas import tpu_sc as plsc

N_NODES = 10000
N_EDGES = 320000
D = 128

NC = 2
NS = 16
CHUNK = 80
EDGES_PER_SC = N_EDGES // NC
EDGES_PER_TILE = EDGES_PER_SC // NS
N_CHUNKS = EDGES_PER_TILE // CHUNK
N_PAD = 10240
ROWS_PER_TILE = N_PAD // NS


NBUF = 4
N_MAIN = (N_CHUNKS - 1) // NBUF * NBUF
N_GROUPS = N_MAIN // NBUF


@functools.partial(
    pl.kernel,
    out_type=jax.ShapeDtypeStruct((NC, N_PAD, D), jnp.float32),
    mesh=plsc.VectorSubcoreMesh(core_axis_name="c", subcore_axis_name="s"),
    scratch_types=[
        pltpu.VMEM((NBUF, 2, CHUNK), jnp.int32),
        pltpu.VMEM((NBUF, CHUNK, D), jnp.float32),
        pltpu.VMEM_SHARED((N_PAD, D), jnp.float32),
        pltpu.SemaphoreType.DMA((NBUF,)),
        pltpu.SemaphoreType.DMA((NBUF,)),
        pltpu.SemaphoreType.DMA((NBUF,)),
    ],
)
def _spmm(h_hbm, eflat_hbm, out_hbm,
          idx_v, rows_v, acc_sh, isem, gsem, ssem):
    c = lax.axis_index("c")
    s = lax.axis_index("s")
    w = c * NS + s
    r0 = s * ROWS_PER_TILE

    def idx_start(j, b):
        base = w * EDGES_PER_TILE + j * CHUNK
        pltpu.async_copy(eflat_hbm.at[pl.ds(N_EDGES + base, CHUNK)],
                         idx_v.at[b, 0], isem.at[b])
        pltpu.async_copy(eflat_hbm.at[pl.ds(base, CHUNK)],
                         idx_v.at[b, 1], isem.at[b])

    def idx_wait(j, b):
        base = w * EDGES_PER_TILE + j * CHUNK
        pltpu.make_async_copy(eflat_hbm.at[pl.ds(N_EDGES + base, CHUNK)],
                              idx_v.at[b, 0], isem.at[b]).wait()
        pltpu.make_async_copy(eflat_hbm.at[pl.ds(base, CHUNK)],
                              idx_v.at[b, 1], isem.at[b]).wait()

    def gather_start(b):
        pltpu.async_copy(h_hbm.at[idx_v.at[b, 0]], rows_v.at[b], gsem.at[b])

    def gather_wait(b):
        pltpu.make_async_copy(h_hbm.at[idx_v.at[b, 0]], rows_v.at[b],
                              gsem.at[b]).wait()

    def scatter_start(b):
        pltpu.async_copy(rows_v.at[b], acc_sh.at[idx_v.at[b, 1]], ssem.at[b],
                         add=True)

    def scatter_wait(b):
        pltpu.make_async_copy(rows_v.at[b], acc_sh.at[idx_v.at[b, 1]],
                              ssem.at[b]).wait()

    for b in range(NBUF):
        idx_start(b, b)
    for i in range(CHUNK):
        for k in range(D // 16):
            rows_v[0, i, pl.ds(k * 16, 16)] = jnp.zeros((16,), jnp.float32)
    zcopies = [
        pltpu.make_async_copy(
            rows_v.at[0], acc_sh.at[pl.ds(r0 + t * CHUNK, CHUNK)], ssem.at[0])
        for t in range(ROWS_PER_TILE // CHUNK)
    ]
    for zc in zcopies:
        zc.start()
    for zc in zcopies:
        zc.wait()
    plsc.subcore_barrier()

    def group(g, carry):
        for b in range(NBUF):
            idx_wait(g * NBUF + b, b)
            gather_start(b)
        for b in range(NBUF):
            gather_wait(b)
            scatter_start(b)
        for b in range(NBUF):
            scatter_wait(b)
            idx_start(g * NBUF + b + NBUF, b)
        return carry

    lax.fori_loop(0, N_GROUPS - 1, group, 0)

    gl = N_GROUPS - 1
    for b in range(NBUF):
        idx_wait(gl * NBUF + b, b)
        gather_start(b)
    for b in range(NBUF):
        gather_wait(b)
        scatter_start(b)
    for b in range(NBUF):
        scatter_wait(b)

    jt = N_CHUNKS - 1
    idx_start(jt, 0)
    idx_wait(jt, 0)
    gather_start(0)
    gather_wait(0)
    scatter_start(0)
    scatter_wait(0)

    plsc.subcore_barrier()
    pltpu.sync_copy(acc_sh.at[pl.ds(r0, ROWS_PER_TILE)],
                    out_hbm.at[c, pl.ds(r0, ROWS_PER_TILE)])


def _mm_body(x_ref, w_ref, o_ref):
    o_ref[...] = jnp.dot(x_ref[...], w_ref[...],
                         preferred_element_type=jnp.float32)


_mm = pl.pallas_call(
    _mm_body,
    out_shape=jax.ShapeDtypeStruct((N_NODES, D), jnp.float32),
)


def _addrelu_body(p_ref, o_ref):
    o_ref[...] = jnp.maximum(p_ref[0] + p_ref[1], 0.0)


_addrelu = pl.pallas_call(
    _addrelu_body,
    out_shape=jax.ShapeDtypeStruct((N_PAD, D), jnp.float32),
)


def _final_body(p_ref, w_ref, o_ref):
    p = p_ref[0, pl.ds(0, N_NODES)] + p_ref[1, pl.ds(0, N_NODES)]
    g = jnp.dot(p, w_ref[...], preferred_element_type=jnp.float32)
    m = jnp.max(g, axis=1, keepdims=True)
    o_ref[...] = (g - m) - jnp.log(jnp.sum(jnp.exp(g - m), axis=1,
                                           keepdims=True))


_final = pl.pallas_call(
    _final_body,
    out_shape=jax.ShapeDtypeStruct((N_NODES, D), jnp.float32),
)


def kernel(x, edge_index, W1, W2):
    eflat = edge_index.reshape(2 * N_EDGES).astype(jnp.int32)
    h1 = _mm(x, W1)
    p1 = _spmm(h1, eflat)
    r = _addrelu(p1)
    p2 = _spmm(r, eflat)
    return _final(p2, W2)


# speedup vs baseline: 1.1206x; 1.0163x over previous
import functools

import jax
import jax.numpy as jnp
from jax import lax
from jax.experimental import pallas as pl
from jax.experimental.pallas import tpu as pltpu
from jax.experimental.pallas import tpu_sc as plsc

N_NODES = 10000
N_EDGES = 320000
D = 128

NC = 2
NS = 16
CHUNK = 80
EDGES_PER_SC = N_EDGES // NC
EDGES_PER_TILE = EDGES_PER_SC // NS
N_CHUNKS = EDGES_PER_TILE // CHUNK
N_PAD = 10240
ROWS_PER_TILE = N_PAD // NS


NBUF = 4
N_MAIN = (N_CHUNKS - 1) // NBUF * NBUF
N_GROUPS = N_MAIN // NBUF


@functools.partial(
    pl.kernel,
    out_type=jax.ShapeDtypeStruct((NC, N_PAD, D), jnp.float32),
    mesh=plsc.VectorSubcoreMesh(core_axis_name="c", subcore_axis_name="s"),
    scratch_types=[
        pltpu.VMEM((NBUF, 2, CHUNK), jnp.int32),
        pltpu.VMEM((NBUF, CHUNK, D), jnp.float32),
        pltpu.VMEM_SHARED((N_PAD, D), jnp.float32),
        pltpu.SemaphoreType.DMA((NBUF,)),
        pltpu.SemaphoreType.DMA((NBUF,)),
        pltpu.SemaphoreType.DMA((NBUF,)),
    ],
)
def _spmm(h_hbm, eflat_hbm, out_hbm,
          idx_v, rows_v, acc_sh, isem, gsem, ssem):
    c = lax.axis_index("c")
    s = lax.axis_index("s")
    w = c * NS + s
    r0 = s * ROWS_PER_TILE

    def idx_start(j, b):
        base = w * EDGES_PER_TILE + j * CHUNK
        pltpu.async_copy(eflat_hbm.at[pl.ds(N_EDGES + base, CHUNK)],
                         idx_v.at[b, 0], isem.at[b])
        pltpu.async_copy(eflat_hbm.at[pl.ds(base, CHUNK)],
                         idx_v.at[b, 1], isem.at[b])

    def idx_wait(j, b):
        base = w * EDGES_PER_TILE + j * CHUNK
        pltpu.make_async_copy(eflat_hbm.at[pl.ds(N_EDGES + base, CHUNK)],
                              idx_v.at[b, 0], isem.at[b]).wait()
        pltpu.make_async_copy(eflat_hbm.at[pl.ds(base, CHUNK)],
                              idx_v.at[b, 1], isem.at[b]).wait()

    def gather_start(b):
        pltpu.async_copy(h_hbm.at[idx_v.at[b, 0]], rows_v.at[b], gsem.at[b])

    def gather_wait(b):
        pltpu.make_async_copy(h_hbm.at[idx_v.at[b, 0]], rows_v.at[b],
                              gsem.at[b]).wait()

    def scatter_start(b):
        pltpu.async_copy(rows_v.at[b], acc_sh.at[idx_v.at[b, 1]], ssem.at[b],
                         add=True)

    def scatter_wait(b):
        pltpu.make_async_copy(rows_v.at[b], acc_sh.at[idx_v.at[b, 1]],
                              ssem.at[b]).wait()

    for b in range(NBUF):
        idx_start(b, b)
    for i in range(CHUNK):
        for k in range(D // 16):
            rows_v[0, i, pl.ds(k * 16, 16)] = jnp.zeros((16,), jnp.float32)
    zcopies = [
        pltpu.make_async_copy(
            rows_v.at[0], acc_sh.at[pl.ds(r0 + t * CHUNK, CHUNK)], ssem.at[0])
        for t in range(ROWS_PER_TILE // CHUNK)
    ]
    for zc in zcopies:
        zc.start()
    for zc in zcopies:
        zc.wait()
    plsc.subcore_barrier()

    def group(g, carry):
        for b in range(NBUF):
            idx_wait(g * NBUF + b, b)
            gather_start(b)
        for b in range(NBUF):
            gather_wait(b)
            scatter_start(b)
        for b in range(NBUF):
            scatter_wait(b)
            idx_start(g * NBUF + b + NBUF, b)
        return carry

    lax.fori_loop(0, N_GROUPS - 1, group, 0)

    gl = N_GROUPS - 1
    for b in range(NBUF):
        idx_wait(gl * NBUF + b, b)
        gather_start(b)
    for b in range(NBUF):
        gather_wait(b)
        scatter_start(b)
    for b in range(NBUF):
        scatter_wait(b)

    jt = N_CHUNKS - 1
    idx_start(jt, 0)
    idx_wait(jt, 0)
    gather_start(0)
    gather_wait(0)
    scatter_start(0)
    scatter_wait(0)

    plsc.subcore_barrier()
    pltpu.sync_copy(acc_sh.at[pl.ds(r0, ROWS_PER_TILE)],
                    out_hbm.at[c, pl.ds(r0, ROWS_PER_TILE)])


def _mm_relu_body(p_ref, w_ref, o_ref):
    o_ref[...] = jnp.maximum(
        jnp.dot(p_ref[0] + p_ref[1], w_ref[...],
                preferred_element_type=jnp.float32), 0.0)


_mm_relu = pl.pallas_call(
    _mm_relu_body,
    out_shape=jax.ShapeDtypeStruct((N_PAD, D), jnp.float32),
)


def _final_body(p_ref, w_ref, o_ref):
    p = p_ref[0, pl.ds(0, N_NODES)] + p_ref[1, pl.ds(0, N_NODES)]
    g = jnp.dot(p, w_ref[...], preferred_element_type=jnp.float32)
    m = jnp.max(g, axis=1, keepdims=True)
    o_ref[...] = (g - m) - jnp.log(jnp.sum(jnp.exp(g - m), axis=1,
                                           keepdims=True))


_final = pl.pallas_call(
    _final_body,
    out_shape=jax.ShapeDtypeStruct((N_NODES, D), jnp.float32),
)


def kernel(x, edge_index, W1, W2):
    eflat = edge_index.reshape(2 * N_EDGES).astype(jnp.int32)
    p1 = _spmm(x, eflat)
    r = _mm_relu(p1, W1)
    p2 = _spmm(r, eflat)
    return _final(p2, W2)
